# baseline (speedup 1.0000x reference)
import jax
import jax.numpy as jnp
from jax import lax
from jax.experimental import pallas as pl
from jax.experimental.pallas import tpu as pltpu
from jax.experimental.pallas import tpu_sc as plsc


def _take16(a, idx):
    return lax.gather(
        a, idx[:, None],
        dimension_numbers=lax.GatherDimensionNumbers(
            offset_dims=(), collapsed_slice_dims=(0,), start_index_map=(0,)),
        slice_sizes=(1,),
        mode=lax.GatherScatterMode.PROMISE_IN_BOUNDS)


M = 100000
D = 128
B = 16384
CAP = 3200
CAP_LAST = M - 31 * CAP
LISTPAD = 160
KSHIFT = 14
CROWS = 80
NVREG = B // 16


def _body(bimg, lab_r, rep_r, last_r, keys, vals, xs, ys, oimg,
          keys_v, vals_v, y_v, win_v, lab_v, rep_v, last_v, dl_v, vl_v,
          row_v, cbuf, sem_in, sem_in2, sem_cg, sem_cs, sem_g, sem_s):
    cid = lax.axis_index("c")
    sid = lax.axis_index("s")
    wid = sid * 2 + cid
    base = wid * CAP
    lanes = lax.iota(jnp.int32, 16)
    neg1 = jnp.full((16,), -1, jnp.int32)
    zero16 = jnp.zeros((16,), jnp.int32)

    def work(cap):
        nchunk = cap // CROWS
        scan_per = NVREG // nchunk
        scan_tail = NVREG - nchunk * scan_per

        with jax.named_scope("ph_pre"):
            kcp = pltpu.async_copy(keys, keys_v, sem_in)
            stage2 = [
                pltpu.async_copy(vals, vals_v, sem_in2),
                pltpu.async_copy(ys, y_v, sem_in2),
                pltpu.async_copy(lab_r.at[pl.ds(base, cap)],
                                 lab_v.at[pl.ds(0, cap)], sem_in2),
                pltpu.async_copy(rep_r.at[pl.ds(base, cap)],
                                 rep_v.at[pl.ds(0, cap)], sem_in2),
                pltpu.async_copy(last_r.at[pl.ds(base, cap)],
                                 last_v.at[pl.ds(0, cap)], sem_in2),
            ]
            pltpu.async_copy(bimg.at[pl.ds(base, CROWS)], cbuf.at[0], sem_cg)
            pltpu.async_copy(bimg.at[pl.ds(base + CROWS, CROWS)], cbuf.at[1],
                             sem_cg)
            kcp.wait()

            def init_body(j, c):
                win_v[pl.ds(j * 16, 16)] = neg1
                return c
            lax.fori_loop(0, cap // 16, init_body, 0)

        def scan_body(j, c):
            k = keys_v[pl.ds(j * 16, 16)]
            comb = (k << KSHIFT) | (lanes + j * 16)
            sc = lax.sort(comb, dimension=0)
            sk = sc >> KSHIFT
            sp = sc & ((1 << KSHIFT) - 1)
            nxt = _take16(sk, jnp.minimum(lanes + 1, 15))
            keep = (nxt != sk) | (lanes == 15)
            inr = (sk >= base) & (sk < base + cap)
            valid = keep & inr
            loc = jnp.where(valid, sk - base, 0)
            cur = plsc.load_gather(win_v, [loc])
            plsc.store_scatter(win_v, [loc], jnp.maximum(cur, sp), mask=valid)
            return c

        def _drain(sem):
            pltpu.make_async_copy(bimg.at[pl.ds(base, CROWS)],
                                  cbuf.at[0], sem).wait()

        def outer_body(s, c):
            def inner(t, c2):
                return scan_body(s * scan_per + t, c2)
            lax.fori_loop(0, scan_per, inner, 0)
            b = lax.rem(s, 4)
            _drain(sem_cg)
            pltpu.async_copy(cbuf.at[b], oimg.at[pl.ds(base + s * CROWS,
                                                       CROWS)], sem_cs)

            @pl.when(s + 2 < nchunk)
            def _():
                @pl.when(s >= 2)
                def _():
                    _drain(sem_cs)
                pltpu.async_copy(bimg.at[pl.ds(base + (s + 2) * CROWS,
                                               CROWS)],
                                 cbuf.at[lax.rem(s + 2, 4)], sem_cg)
            return c
        with jax.named_scope("ph_scan_copy"):
            lax.fori_loop(0, nchunk, outer_body, 0)

            def tail(t, c2):
                return scan_body(nchunk * scan_per + t, c2)
            lax.fori_loop(0, scan_tail, tail, 0)

        with jax.named_scope("ph_mid"):
            for c in stage2:
                c.wait()

        def comp_body(j, cnt):
            w = win_v[pl.ds(j * 16, 16)]
            m = w >= 0
            r = lanes + j * 16
            wsafe = jnp.where(m, w, 0)
            val = plsc.load_gather(vals_v, [wsafe])
            lab = plsc.load_gather(y_v, [jnp.where(m, val, 0)])
            plsc.store_scatter(lab_v, [r], lab, mask=m)
            plsc.store_scatter(rep_v, [r], zero16, mask=m)
            plsc.store_scatter(last_v, [r], zero16, mask=m)
            plsc.store_compressed(dl_v.at[pl.ds(cnt, 16)], r + base, mask=m)
            plsc.store_compressed(vl_v.at[pl.ds(cnt, 16)], val, mask=m)
            return cnt + jnp.sum(jnp.where(m, 1, 0))
        with jax.named_scope("ph_compact"):
            cnt = lax.fori_loop(0, cap // 16, comp_body, jnp.int32(0))

        @pl.when(cnt > 0)
        def _pad():
            zi = jnp.zeros((16,), jnp.int32)
            d0 = _take16(dl_v[pl.ds(0, 16)], zi)
            v0 = _take16(vl_v[pl.ds(0, 16)], zi)

            def pad_body(p, c):
                idxs = cnt + p * 16 + lanes
                plsc.store_scatter(dl_v, [idxs], d0)
                plsc.store_scatter(vl_v, [idxs], v0)
                return c
            lax.fori_loop(0, 8, pad_body, 0)

        _drain(sem_cs)
        _drain(sem_cs)
        _drain(sem_cs)
        _drain(sem_cs)

        nch4 = ((cnt + 63) // 64) * 4

        def _fire_g(kk):
            vvec = vl_v[pl.ds(kk * 16, 16)]
            pltpu.async_copy(xs.at[vvec], row_v.at[lax.rem(kk, 8)], sem_g)

        def _drain_row(sem):
            pltpu.make_async_copy(xs.at[pl.ds(0, 16)], row_v.at[0],
                                  sem).wait()

        def chunk_body(kk, c):
            _drain_row(sem_g)
            dvec = dl_v[pl.ds(kk * 16, 16)]
            pltpu.async_copy(row_v.at[lax.rem(kk, 8)], oimg.at[dvec], sem_s)

            @pl.when(kk >= 4)
            def _():
                _drain_row(sem_s)

            @pl.when(kk + 4 < nch4)
            def _():
                _fire_g(kk + 4)
            return c

        with jax.named_scope("ph_rowdma"):
            @pl.when(cnt > 0)
            def _rowdma():
                for p in range(4):
                    _fire_g(p)
                lax.fori_loop(0, nch4, chunk_body, 0)
                for p in range(4):
                    _drain_row(sem_s)

        with jax.named_scope("ph_stage_out"):
            pltpu.sync_copy(lab_v.at[pl.ds(0, cap)],
                            lab_r.at[pl.ds(base, cap)])
            pltpu.sync_copy(rep_v.at[pl.ds(0, cap)],
                            rep_r.at[pl.ds(base, cap)])
            pltpu.sync_copy(last_v.at[pl.ds(0, cap)],
                            last_r.at[pl.ds(base, cap)])

    @pl.when(wid < 31)
    def _full():
        work(CAP)

    @pl.when(wid == 31)
    def _last():
        work(CAP_LAST)


_mesh = plsc.VectorSubcoreMesh(core_axis_name="c", subcore_axis_name="s")

_sc_overwrite = pl.kernel(
    _body,
    out_type=(jax.ShapeDtypeStruct((M, D), jnp.float32),),
    mesh=_mesh,
    compiler_params=pltpu.CompilerParams(needs_layout_passes=False),
    scratch_types=(
        pltpu.VMEM((B,), jnp.int32),
        pltpu.VMEM((B,), jnp.int32),
        pltpu.VMEM((B,), jnp.int32),
        pltpu.VMEM((CAP,), jnp.int32),
        pltpu.VMEM((CAP,), jnp.int32),
        pltpu.VMEM((CAP,), jnp.int32),
        pltpu.VMEM((CAP,), jnp.int32),
        pltpu.VMEM((CAP + LISTPAD,), jnp.int32),
        pltpu.VMEM((CAP + LISTPAD,), jnp.int32),
        pltpu.VMEM((8, 16, D), jnp.float32),
        pltpu.VMEM((4, CROWS, D), jnp.float32),
        pltpu.SemaphoreType.DMA,
        pltpu.SemaphoreType.DMA,
        pltpu.SemaphoreType.DMA,
        pltpu.SemaphoreType.DMA,
        pltpu.SemaphoreType.DMA,
        pltpu.SemaphoreType.DMA,
    ),
)


def kernel(buffer_img, buffer_label, buffer_replay_times, buffer_last_replay,
           idx_keys, idx_vals, x, y):
    lab_ref = jax.new_ref(buffer_label.astype(jnp.int32))
    rep_ref = jax.new_ref(buffer_replay_times.astype(jnp.int32))
    last_ref = jax.new_ref(buffer_last_replay.astype(jnp.int32))
    out_img, = _sc_overwrite(buffer_img, lab_ref, rep_ref, last_ref,
                             idx_keys.astype(jnp.int32),
                             idx_vals.astype(jnp.int32),
                             x,
                             y.astype(jnp.int32))
    return (out_img,
            jax.freeze(lab_ref).astype(buffer_label.dtype),
            jax.freeze(rep_ref).astype(buffer_replay_times.dtype),
            jax.freeze(last_ref).astype(buffer_last_replay.dtype))


# speedup vs baseline: 1.0299x; 1.0299x over previous
import jax
import jax.numpy as jnp
from jax import lax
from jax.experimental import pallas as pl
from jax.experimental.pallas import tpu as pltpu
from jax.experimental.pallas import tpu_sc as plsc


def _take16(a, idx):
    return lax.gather(
        a, idx[:, None],
        dimension_numbers=lax.GatherDimensionNumbers(
            offset_dims=(), collapsed_slice_dims=(0,), start_index_map=(0,)),
        slice_sizes=(1,),
        mode=lax.GatherScatterMode.PROMISE_IN_BOUNDS)


M = 100000
D = 128
B = 16384
CAP = 3200
CAP_LAST = M - 31 * CAP
LISTPAD = 160
KSHIFT = 14
CROWS = 80
NVREG = B // 16


def _body(bimg, lab_r, rep_r, last_r, keys, vals, xs, ys, oimg,
          keys_v, vals_v, y_v, win_v, lab_v, rep_v, last_v, dl_v, vl_v,
          cbuf, sem_in, sem_in2, sem_cg, sem_cs, sem_g, sem_s):
    cid = lax.axis_index("c")
    sid = lax.axis_index("s")
    wid = sid * 2 + cid
    base = wid * CAP
    lanes = lax.iota(jnp.int32, 16)
    neg1 = jnp.full((16,), -1, jnp.int32)
    zero16 = jnp.zeros((16,), jnp.int32)

    def work(cap):
        nchunk = cap // CROWS
        scan_per = NVREG // nchunk
        scan_tail = NVREG - nchunk * scan_per

        with jax.named_scope("ph_pre"):
            kcp = pltpu.async_copy(keys, keys_v, sem_in)
            stage2 = [
                pltpu.async_copy(vals, vals_v, sem_in2),
                pltpu.async_copy(ys, y_v, sem_in2),
                pltpu.async_copy(lab_r.at[pl.ds(base, cap)],
                                 lab_v.at[pl.ds(0, cap)], sem_in2),
                pltpu.async_copy(rep_r.at[pl.ds(base, cap)],
                                 rep_v.at[pl.ds(0, cap)], sem_in2),
                pltpu.async_copy(last_r.at[pl.ds(base, cap)],
                                 last_v.at[pl.ds(0, cap)], sem_in2),
            ]
            pltpu.async_copy(bimg.at[pl.ds(base, CROWS)], cbuf.at[0], sem_cg)
            pltpu.async_copy(bimg.at[pl.ds(base + CROWS, CROWS)], cbuf.at[1],
                             sem_cg)
            kcp.wait()

            def init_body(j, c):
                win_v[pl.ds(j * 16, 16)] = neg1
                return c
            lax.fori_loop(0, cap // 16, init_body, 0)

        def scan_body(j, c):
            k = keys_v[pl.ds(j * 16, 16)]
            comb = (k << KSHIFT) | (lanes + j * 16)
            sc = lax.sort(comb, dimension=0)
            sk = sc >> KSHIFT
            sp = sc & ((1 << KSHIFT) - 1)
            nxt = _take16(sk, jnp.minimum(lanes + 1, 15))
            keep = (nxt != sk) | (lanes == 15)
            inr = (sk >= base) & (sk < base + cap)
            valid = keep & inr
            loc = jnp.where(valid, sk - base, 0)
            cur = plsc.load_gather(win_v, [loc])
            plsc.store_scatter(win_v, [loc], jnp.maximum(cur, sp), mask=valid)
            return c

        def _drain(sem):
            pltpu.make_async_copy(bimg.at[pl.ds(base, CROWS)],
                                  cbuf.at[0], sem).wait()

        def outer_body(s, c):
            def inner(t, c2):
                return scan_body(s * scan_per + t, c2)
            lax.fori_loop(0, scan_per, inner, 0)
            b = lax.rem(s, 4)
            _drain(sem_cg)
            pltpu.async_copy(cbuf.at[b], oimg.at[pl.ds(base + s * CROWS,
                                                       CROWS)], sem_cs)

            @pl.when(s + 2 < nchunk)
            def _():
                @pl.when(s >= 2)
                def _():
                    _drain(sem_cs)
                pltpu.async_copy(bimg.at[pl.ds(base + (s + 2) * CROWS,
                                               CROWS)],
                                 cbuf.at[lax.rem(s + 2, 4)], sem_cg)
            return c
        with jax.named_scope("ph_scan_copy"):
            lax.fori_loop(0, nchunk, outer_body, 0)

            def tail(t, c2):
                return scan_body(nchunk * scan_per + t, c2)
            lax.fori_loop(0, scan_tail, tail, 0)

        with jax.named_scope("ph_mid"):
            for c in stage2:
                c.wait()

        def comp_body(j, cnt):
            w = win_v[pl.ds(j * 16, 16)]
            m = w >= 0
            r = lanes + j * 16
            wsafe = jnp.where(m, w, 0)
            val = plsc.load_gather(vals_v, [wsafe])
            lab = plsc.load_gather(y_v, [jnp.where(m, val, 0)])
            plsc.store_scatter(lab_v, [r], lab, mask=m)
            plsc.store_scatter(rep_v, [r], zero16, mask=m)
            plsc.store_scatter(last_v, [r], zero16, mask=m)
            plsc.store_compressed(dl_v.at[pl.ds(cnt, 16)], r + base, mask=m)
            plsc.store_compressed(vl_v.at[pl.ds(cnt, 16)], val, mask=m)
            return cnt + jnp.sum(jnp.where(m, 1, 0))
        with jax.named_scope("ph_compact"):
            cnt = lax.fori_loop(0, cap // 16, comp_body, jnp.int32(0))

        @pl.when(cnt > 0)
        def _pad():
            zi = jnp.zeros((16,), jnp.int32)
            d0 = _take16(dl_v[pl.ds(0, 16)], zi)
            v0 = _take16(vl_v[pl.ds(0, 16)], zi)

            def pad_body(p, c):
                idxs = cnt + p * 16 + lanes
                plsc.store_scatter(dl_v, [idxs], d0)
                plsc.store_scatter(vl_v, [idxs], v0)
                return c
            lax.fori_loop(0, 8, pad_body, 0)

        _drain(sem_cs)
        _drain(sem_cs)
        _drain(sem_cs)
        _drain(sem_cs)

        nch4 = jnp.maximum((cnt + 63) // 64, 2) * 4

        def _rbuf(k):
            b = lax.rem(k, 16)
            return cbuf.at[b // 5, pl.ds(lax.rem(b, 5) * 16, 16)]

        def _fire_g(kk):
            vvec = vl_v[pl.ds(kk * 16, 16)]
            pltpu.async_copy(xs.at[vvec], _rbuf(kk), sem_g)

        def _drain_row(sem):
            pltpu.make_async_copy(xs.at[pl.ds(0, 16)], _rbuf(0), sem).wait()

        def chunk_body(kk, c):
            _drain_row(sem_g)
            dvec = dl_v[pl.ds(kk * 16, 16)]
            pltpu.async_copy(_rbuf(kk), oimg.at[dvec], sem_s)

            @pl.when(kk >= 8)
            def _():
                _drain_row(sem_s)

            @pl.when(kk + 8 < nch4)
            def _():
                _fire_g(kk + 8)
            return c

        with jax.named_scope("ph_rowdma"):
            @pl.when(cnt > 0)
            def _rowdma():
                for p in range(8):
                    _fire_g(p)
                lax.fori_loop(0, nch4, chunk_body, 0)
                for p in range(8):
                    _drain_row(sem_s)

        with jax.named_scope("ph_stage_out"):
            pltpu.sync_copy(lab_v.at[pl.ds(0, cap)],
                            lab_r.at[pl.ds(base, cap)])
            pltpu.sync_copy(rep_v.at[pl.ds(0, cap)],
                            rep_r.at[pl.ds(base, cap)])
            pltpu.sync_copy(last_v.at[pl.ds(0, cap)],
                            last_r.at[pl.ds(base, cap)])

    @pl.when(wid < 31)
    def _full():
        work(CAP)

    @pl.when(wid == 31)
    def _last():
        work(CAP_LAST)


_mesh = plsc.VectorSubcoreMesh(core_axis_name="c", subcore_axis_name="s")

_sc_overwrite = pl.kernel(
    _body,
    out_type=(jax.ShapeDtypeStruct((M, D), jnp.float32),),
    mesh=_mesh,
    compiler_params=pltpu.CompilerParams(needs_layout_passes=False),
    scratch_types=(
        pltpu.VMEM((B,), jnp.int32),
        pltpu.VMEM((B,), jnp.int32),
        pltpu.VMEM((B,), jnp.int32),
        pltpu.VMEM((CAP,), jnp.int32),
        pltpu.VMEM((CAP,), jnp.int32),
        pltpu.VMEM((CAP,), jnp.int32),
        pltpu.VMEM((CAP,), jnp.int32),
        pltpu.VMEM((CAP + LISTPAD,), jnp.int32),
        pltpu.VMEM((CAP + LISTPAD,), jnp.int32),
        pltpu.VMEM((4, CROWS, D), jnp.float32),
        pltpu.SemaphoreType.DMA,
        pltpu.SemaphoreType.DMA,
        pltpu.SemaphoreType.DMA,
        pltpu.SemaphoreType.DMA,
        pltpu.SemaphoreType.DMA,
        pltpu.SemaphoreType.DMA,
    ),
)


def kernel(buffer_img, buffer_label, buffer_replay_times, buffer_last_replay,
           idx_keys, idx_vals, x, y):
    lab_ref = jax.new_ref(buffer_label.astype(jnp.int32))
    rep_ref = jax.new_ref(buffer_replay_times.astype(jnp.int32))
    last_ref = jax.new_ref(buffer_last_replay.astype(jnp.int32))
    out_img, = _sc_overwrite(buffer_img, lab_ref, rep_ref, last_ref,
                             idx_keys.astype(jnp.int32),
                             idx_vals.astype(jnp.int32),
                             x,
                             y.astype(jnp.int32))
    return (out_img,
            jax.freeze(lab_ref).astype(buffer_label.dtype),
            jax.freeze(rep_ref).astype(buffer_replay_times.dtype),
            jax.freeze(last_ref).astype(buffer_last_replay.dtype))
